# baseline (speedup 1.0000x reference)
import jax
import jax.numpy as jnp
from jax.experimental import pallas as pl
from jax.experimental.pallas import tpu as pltpu

_T = 8


def _band_avg_kernel(band_ref, w_ref, x_ref, o_ref):
    o_ref[...] = jnp.zeros(o_ref.shape, o_ref.dtype)

    def body(f, _):
        g = band_ref[f]
        wf = w_ref[f]
        o_ref[:, pl.ds(g, 1)] += x_ref[:, pl.ds(f, 1)] * wf
        return 0

    jax.lax.fori_loop(0, x_ref.shape[1], body, 0)


def kernel(x, freq_masks):
    b, t, f, c1, c2 = x.shape
    g = freq_masks.shape[0]
    bt = b * t
    xr = x.reshape(bt, f, c1, c2)

    counts = jnp.sum(freq_masks, axis=1)
    w = jnp.sum(freq_masks / counts[:, None], axis=0)
    band_idx = jnp.argmax(freq_masks, axis=0).astype(jnp.int32)

    grid = (bt // _T,)
    out = pl.pallas_call(
        _band_avg_kernel,
        out_shape=jax.ShapeDtypeStruct((bt, g, c1, c2), jnp.float32),
        grid_spec=pltpu.PrefetchScalarGridSpec(
            num_scalar_prefetch=2,
            grid=grid,
            in_specs=[
                pl.BlockSpec((_T, f, c1, c2), lambda i, *_: (i, 0, 0, 0)),
            ],
            out_specs=pl.BlockSpec((_T, g, c1, c2), lambda i, *_: (i, 0, 0, 0)),
        ),
        compiler_params=pltpu.CompilerParams(
            dimension_semantics=("parallel",),
            vmem_limit_bytes=56 * 1024 * 1024,
        ),
        name="freq_band_avg",
    )(band_idx, w, xr)

    return out.reshape(b, t, g, c1, c2)


# speedup vs baseline: 3.0064x; 3.0064x over previous
import jax
import jax.numpy as jnp
from jax.experimental import pallas as pl
from jax.experimental.pallas import tpu as pltpu

_T = 8


def _band_avg_kernel(m_ref, x_ref, o_ref):
    m = m_ref[...]
    for r in range(_T):
        o_ref[r] = jax.lax.dot_general(
            m, x_ref[r],
            dimension_numbers=(((1,), (1,)), ((), ())),
            preferred_element_type=jnp.float32,
        )


def kernel(x, freq_masks):
    b, t, f, c1, c2 = x.shape
    g = freq_masks.shape[0]
    bt = b * t
    c = c1 * c2

    xr = x.transpose(0, 1, 3, 4, 2).reshape(bt, c, f)

    counts = jnp.sum(freq_masks, axis=1, keepdims=True)
    sm = freq_masks / counts

    grid = (bt // _T,)
    out = pl.pallas_call(
        _band_avg_kernel,
        out_shape=jax.ShapeDtypeStruct((bt, g, c), jnp.float32),
        grid=grid,
        in_specs=[
            pl.BlockSpec((g, f), lambda i: (0, 0)),
            pl.BlockSpec((_T, c, f), lambda i: (i, 0, 0)),
        ],
        out_specs=pl.BlockSpec((_T, g, c), lambda i: (i, 0, 0)),
        compiler_params=pltpu.CompilerParams(
            dimension_semantics=("parallel",),
            vmem_limit_bytes=56 * 1024 * 1024,
        ),
        name="freq_band_avg",
    )(sm, xr)

    return out.reshape(b, t, g, c1, c2)


# speedup vs baseline: 3.8472x; 1.2797x over previous
import jax
import jax.numpy as jnp
from jax.experimental import pallas as pl
from jax.experimental.pallas import tpu as pltpu

_T = 8
_TP = 56


def _band_avg_kernel(m_ref, x_ref, o_ref, *, n_t):
    m = m_ref[...]
    j = pl.program_id(1)
    for r in range(_T):
        valid = (j * _T + r) < n_t
        xa = jnp.where(valid, x_ref[0, r], 0.0)
        o_ref[:, 0, r, :] = jax.lax.dot_general(
            m, xa,
            dimension_numbers=(((1,), (1,)), ((), ())),
            preferred_element_type=jnp.float32,
        )


def _t_to_lanes_kernel(y_ref, eye_ref, o_ref):
    nt = o_ref.shape[4]
    for g in range(o_ref.shape[1]):
        s = y_ref[g, 0]
        st = jax.lax.dot_general(
            s, eye_ref[...],
            dimension_numbers=(((0,), (0,)), ((), ())),
            preferred_element_type=jnp.float32,
        )
        o_ref[0, g] = st[:, :nt].reshape(
            o_ref.shape[2], o_ref.shape[3], nt)


def kernel(x, freq_masks):
    b, t, f, c1, c2 = x.shape
    g = freq_masks.shape[0]
    c = c1 * c2

    xr = x.transpose(0, 1, 3, 4, 2).reshape(b, t, c, f)

    counts = jnp.sum(freq_masks, axis=1, keepdims=True)
    sm = freq_masks / counts

    import functools
    band = pl.pallas_call(
        functools.partial(_band_avg_kernel, n_t=t),
        out_shape=jax.ShapeDtypeStruct((g, b, _TP, c), jnp.float32),
        grid=(b, _TP // _T),
        in_specs=[
            pl.BlockSpec((g, f), lambda i, j: (0, 0)),
            pl.BlockSpec((1, _T, c, f), lambda i, j: (i, j, 0, 0)),
        ],
        out_specs=pl.BlockSpec((g, 1, _T, c), lambda i, j: (0, i, j, 0)),
        compiler_params=pltpu.CompilerParams(
            dimension_semantics=("parallel", "arbitrary"),
            vmem_limit_bytes=56 * 1024 * 1024,
        ),
        name="freq_band_avg",
    )(sm, xr)

    eye = jnp.eye(_TP, dtype=jnp.float32)

    out5 = pl.pallas_call(
        _t_to_lanes_kernel,
        out_shape=jax.ShapeDtypeStruct((b, g, c1, c2, t), jnp.float32),
        grid=(b,),
        in_specs=[
            pl.BlockSpec((g, 1, _TP, c), lambda i: (0, i, 0, 0)),
            pl.BlockSpec((_TP, _TP), lambda i: (0, 0)),
        ],
        out_specs=pl.BlockSpec((1, g, c1, c2, t), lambda i: (i, 0, 0, 0, 0)),
        compiler_params=pltpu.CompilerParams(
            dimension_semantics=("arbitrary",),
            vmem_limit_bytes=56 * 1024 * 1024,
        ),
        name="bands_t_to_lanes",
    )(band, eye)

    return out5.transpose(0, 4, 1, 2, 3)


# speedup vs baseline: 3.9271x; 1.0208x over previous
import functools

import jax
import jax.numpy as jnp
from jax.experimental import pallas as pl
from jax.experimental.pallas import tpu as pltpu

_T = 8
_NCHUNK = 7


def _fused_kernel(m_ref, x_ref, eye_ref, o_ref, acc_ref, *, n_t):
    j = pl.program_id(1)

    @pl.when(j < _NCHUNK)
    def _():
        m = m_ref[...]
        for r in range(_T):
            valid = (j * _T + r) < n_t
            xa = jnp.where(valid, x_ref[0, r], 0.0)
            acc_ref[pl.ds(j, 1), :, r, :] = jax.lax.dot_general(
                m, xa,
                dimension_numbers=(((1,), (1,)), ((), ())),
                preferred_element_type=jnp.float32,
            )[None]

    @pl.when(j == _NCHUNK)
    def _():
        for g in range(o_ref.shape[1]):
            s = acc_ref[:, g].reshape(_NCHUNK * _T, acc_ref.shape[3])
            st = jax.lax.dot_general(
                s, eye_ref[...],
                dimension_numbers=(((0,), (0,)), ((), ())),
                preferred_element_type=jnp.float32,
            )
            o_ref[0, g] = st[:, :n_t].reshape(
                o_ref.shape[2], o_ref.shape[3], n_t)


def kernel(x, freq_masks):
    b, t, f, c1, c2 = x.shape
    g = freq_masks.shape[0]
    c = c1 * c2

    xr = x.transpose(0, 1, 3, 4, 2).reshape(b, t, c, f)

    counts = jnp.sum(freq_masks, axis=1, keepdims=True)
    sm = freq_masks / counts
    eye = jnp.eye(_NCHUNK * _T, dtype=jnp.float32)

    out5 = pl.pallas_call(
        functools.partial(_fused_kernel, n_t=t),
        out_shape=jax.ShapeDtypeStruct((b, g, c1, c2, t), jnp.float32),
        grid=(b, _NCHUNK + 1),
        in_specs=[
            pl.BlockSpec((g, f), lambda i, j: (0, 0)),
            pl.BlockSpec((1, _T, c, f),
                         lambda i, j: (i, jnp.minimum(j, _NCHUNK - 1), 0, 0)),
            pl.BlockSpec((_NCHUNK * _T, _NCHUNK * _T), lambda i, j: (0, 0)),
        ],
        out_specs=pl.BlockSpec((1, g, c1, c2, t),
                               lambda i, j: (i, 0, 0, 0, 0)),
        scratch_shapes=[pltpu.VMEM((_NCHUNK, g, _T, c), jnp.float32)],
        compiler_params=pltpu.CompilerParams(
            dimension_semantics=("parallel", "arbitrary"),
            vmem_limit_bytes=56 * 1024 * 1024,
        ),
        name="freq_band_avg",
    )(sm, xr, eye)

    return out5.transpose(0, 4, 1, 2, 3)
